# Initial kernel scaffold; baseline (speedup 1.0000x reference)
import jax
import jax.numpy as jnp
from jax.experimental import pallas as pl


def kernel(x, y, adj_val, adjlg_val, p_val, deg, deg_lg, W_alpha, W_beta, W_alpha_lg, W_beta_lg, bn_gamma, bn_beta, adj_idx, adjlg_idx, p_rows, p_cols):
    raise NotImplementedError("write your pallas kernel here")


# speedup vs baseline: 1.0034x; 1.0034x over previous
import functools

import jax
import jax.numpy as jnp
from jax import lax
from jax.experimental import pallas as pl
from jax.experimental.pallas import tpu as pltpu

EPS = 1e-3
D = 128


def _dense_body(x_ref, ax_ref, aax_ref, py_ref, w0_ref, w3_ref, w4_ref,
                w5_ref, scale_ref, bias_ref, o_ref):
    h = jnp.dot(x_ref[...], w0_ref[...], preferred_element_type=jnp.float32)
    h += jnp.dot(ax_ref[...], w3_ref[...], preferred_element_type=jnp.float32)
    h += jnp.dot(aax_ref[...], w4_ref[...], preferred_element_type=jnp.float32)
    h += jnp.dot(py_ref[...], w5_ref[...], preferred_element_type=jnp.float32)
    col = lax.broadcasted_iota(jnp.int32, h.shape, 1)
    h = jnp.where(col < D, jnp.maximum(h, 0.0), h)
    o_ref[...] = h * scale_ref[...] + bias_ref[...]


def _dense_stage(x, ax, aax, py, w0, w3, w4, w5, scale, bias, blk):
    n = x.shape[0]
    grid = (n + blk - 1) // blk
    row_spec = pl.BlockSpec((blk, D), lambda i: (i, 0))
    w_spec = pl.BlockSpec((D, 2 * D), lambda i: (0, 0))
    v_spec = pl.BlockSpec((1, 2 * D), lambda i: (0, 0))
    return pl.pallas_call(
        _dense_body,
        grid=(grid,),
        in_specs=[row_spec, row_spec, row_spec, row_spec,
                  w_spec, w_spec, w_spec, w_spec, v_spec, v_spec],
        out_specs=pl.BlockSpec((blk, 2 * D), lambda i: (i, 0)),
        out_shape=jax.ShapeDtypeStruct((n, 2 * D), jnp.float32),
    )(x, ax, aax, py, w0, w3, w4, w5, scale, bias)


def _spmm(vals, rows, cols, X, n_out):
    return jnp.zeros((n_out, X.shape[1]), X.dtype).at[rows].add(
        vals[:, None] * X[cols])


def kernel(x, y, adj_val, adjlg_val, p_val, deg, deg_lg, W_alpha, W_beta,
           W_alpha_lg, W_beta_lg, bn_gamma, bn_beta, adj_idx, adjlg_idx,
           p_rows, p_cols):
    N = x.shape[0]
    M = y.shape[0]

    def cat(wa, wb):
        return jnp.concatenate((wa, wb), axis=1)

    w0_n = cat(W_alpha[0] + W_alpha[1] + W_alpha[2],
               W_beta[0] + W_beta[1] + W_beta[2])
    w3_n = cat(W_alpha[3], W_beta[3])
    w4_n = cat(W_alpha[4], W_beta[4])
    w5_n = cat(W_alpha[5], W_beta[5])
    w0_l = cat(W_alpha_lg[0] + W_alpha_lg[1] + W_alpha_lg[2],
               W_beta_lg[0] + W_beta_lg[1] + W_beta_lg[2])
    w3_l = cat(W_alpha_lg[3], W_beta_lg[3])
    w4_l = cat(W_alpha_lg[4], W_beta_lg[4])
    w5_l = w5_n

    inv = 1.0 / jnp.sqrt(1.0 + EPS)
    scale_n = jnp.concatenate((bn_gamma[0], bn_gamma[1]))[None, :] * inv
    bias_n = jnp.concatenate((bn_beta[0], bn_beta[1]))[None, :]
    scale_l = jnp.concatenate((bn_gamma[2], bn_gamma[3]))[None, :] * inv
    bias_l = jnp.concatenate((bn_beta[2], bn_beta[3]))[None, :]

    ax = _spmm(adj_val, adj_idx[0], adj_idx[1], x, N)
    aax = _spmm(adj_val, adj_idx[0], adj_idx[1], ax, N)
    aly = _spmm(adjlg_val, adjlg_idx[0], adjlg_idx[1], y, M)
    aaly = _spmm(adjlg_val, adjlg_idx[0], adjlg_idx[1], aly, M)
    py = _spmm(p_val, p_rows, p_cols, y, N)
    px = _spmm(p_val, p_cols, p_rows, x, M)

    out_n = _dense_stage(x, ax, aax, py, w0_n, w3_n, w4_n, w5_n,
                         scale_n, bias_n, blk=512)
    out_l = _dense_stage(y, aly, aaly, px, w0_l, w3_l, w4_l, w5_l,
                         scale_l, bias_l, blk=512)
    return (out_n, out_l)


# speedup vs baseline: 1.2180x; 1.2139x over previous
import functools

import jax
import jax.numpy as jnp
from jax import lax
from jax.experimental import pallas as pl
from jax.experimental.pallas import tpu as pltpu
from jax.experimental.pallas import tpu_sc as plsc

EPS = 1e-3
D = 128
B = 2000
G = 128
CAP = 8192
NSUB = 16
NCORE = 2


_DNUMS = lax.GatherDimensionNumbers(
    offset_dims=(), collapsed_slice_dims=(0,), start_index_map=(0,))


def _vgather(v, idx):
    return lax.gather(v, idx[:, None], _DNUMS, (1,),
                      mode=lax.GatherScatterMode.PROMISE_IN_BOUNDS)


def _lane_bcast(v, j):
    return _vgather(v, jnp.full((16,), j, dtype=jnp.int32))


@functools.cache
def _make_spmm(E, n_out, n_in):
    R = 2000 if n_out <= 10000 else 8000
    C = n_out // R
    assert C * R == n_out
    CPC = (C + NCORE - 1) // NCORE
    E_w = E // NSUB
    NB = E_w // B
    assert NB * B == E_w
    base = (R // NSUB) // 8 * 8
    last = R - (NSUB - 1) * base
    assert last % 8 == 0 and 0 < last

    mesh = plsc.VectorSubcoreMesh(core_axis_name="c", subcore_axis_name="s",
                                  num_cores=NCORE, num_subcores=NSUB)

    @functools.partial(
        pl.kernel,
        out_type=jax.ShapeDtypeStruct((n_out, D), jnp.float32),
        mesh=mesh,
        scratch_types=[
            pltpu.VMEM_SHARED((R, D), jnp.float32),
            pltpu.VMEM((B,), jnp.int32),
            pltpu.VMEM((B,), jnp.int32),
            pltpu.VMEM((B,), jnp.float32),
            pltpu.VMEM((CAP,), jnp.int32),
            pltpu.VMEM((CAP,), jnp.int32),
            pltpu.VMEM((CAP,), jnp.float32),
            pltpu.VMEM((G,), jnp.int32),
            pltpu.VMEM((G,), jnp.int32),
            pltpu.VMEM((G,), jnp.float32),
            pltpu.VMEM((G, D), jnp.float32),
            pltpu.SemaphoreType.DMA,
        ],
    )
    def spmm(rows_h, cols_h, vals_h, x_h, z_h, out_h, acc, rbuf, cbuf, vbuf,
             crows, ccols, cvals, grow, gcol, gval, gbuf, sem):
        cid = lax.axis_index("c")
        sid = lax.axis_index("s")
        lanes = lax.iota(jnp.int32, 16)

        def do_batch(done, ptr):
            def stage_k(k, _):
                offs = done + k * 16
                valid = (offs + lanes) < ptr
                rr = crows[pl.ds(offs, 16)]
                cc = ccols[pl.ds(offs, 16)]
                vv = cvals[pl.ds(offs, 16)]
                grow[pl.ds(k * 16, 16)] = jnp.where(valid, rr, 0)
                gcol[pl.ds(k * 16, 16)] = jnp.where(valid, cc, 0)
                gval[pl.ds(k * 16, 16)] = jnp.where(valid, vv, 0.0)
                return 0
            lax.fori_loop(0, G // 16, stage_k, 0)
            pltpu.async_copy(x_h.at[gcol], gbuf, sem).wait()

            def scale_g(g, _):
                v16 = gval[pl.ds(g * 16, 16)]
                for j in range(16):
                    sp = _lane_bcast(v16, j)
                    e = g * 16 + j
                    for b in range(D // 16):
                        gbuf[e, pl.ds(b * 16, 16)] = (
                            gbuf[e, pl.ds(b * 16, 16)] * sp)
                return 0
            lax.fori_loop(0, G // 16, scale_g, 0)
            pltpu.sync_copy(gbuf, acc.at[grow], add=True)

        def chunk_body(ci, _):
            chunk = ci * NCORE + cid
            lo = chunk * R

            @pl.when(chunk < C)
            def _chunk_work():
                _run_chunk(lo)
            return 0

        def _run_chunk(lo):
            @pl.when(sid < NSUB - 1)
            def _():
                pltpu.sync_copy(z_h.at[pl.ds(0, base), :],
                                acc.at[pl.ds(sid * base, base), :])

            @pl.when(sid == NSUB - 1)
            def _():
                pltpu.sync_copy(z_h.at[pl.ds(0, last), :],
                                acc.at[pl.ds((NSUB - 1) * base, last), :])
            plsc.subcore_barrier()

            def block_body(blk, carry):
                done, ptr = carry
                eb = pl.multiple_of(sid * E_w + blk * B, 16)
                pltpu.sync_copy(rows_h.at[pl.ds(eb, B)], rbuf)
                pltpu.sync_copy(cols_h.at[pl.ds(eb, B)], cbuf)
                pltpu.sync_copy(vals_h.at[pl.ds(eb, B)], vbuf)

                def scan_i(i, ptr):
                    r = rbuf[pl.ds(i * 16, 16)]
                    c = cbuf[pl.ds(i * 16, 16)]
                    v = vbuf[pl.ds(i * 16, 16)]
                    m = (r >= lo) & (r < lo + R)
                    csum = jnp.where(m, 1, 0)
                    for s in (1, 2, 4, 8):
                        sh = _vgather(csum, jnp.maximum(lanes - s, 0))
                        csum = csum + jnp.where(lanes >= s, sh, 0)
                    target = lanes + 1
                    lo_l = jnp.zeros((16,), jnp.int32)
                    hi_l = jnp.full((16,), 15, jnp.int32)
                    for _ in range(4):
                        mid = lax.shift_right_logical(lo_l + hi_l, 1)
                        ge = _vgather(csum, mid) >= target
                        hi_l = jnp.where(ge, mid, hi_l)
                        lo_l = jnp.where(ge, lo_l, mid + 1)
                    crows[pl.ds(ptr, 16)] = _vgather(r - lo, lo_l)
                    ccols[pl.ds(ptr, 16)] = _vgather(c, lo_l)
                    cvals[pl.ds(ptr, 16)] = _vgather(v, lo_l)
                    cnt = jnp.squeeze(lax.slice(csum, (15,), (16,)))
                    ok = jnp.where(ptr < CAP - 32, 1, 0)
                    return ptr + cnt * ok
                ptr = lax.fori_loop(0, B // 16, scan_i, ptr)

                def flush_k(k, c):
                    d, p = c
                    cond = d + G <= p

                    @pl.when(cond)
                    def _():
                        do_batch(d, p)
                    return d + jnp.where(cond, G, 0), p
                done, ptr = lax.fori_loop(0, (B + G - 1) // G + 1, flush_k,
                                          (done, ptr))
                return done, ptr

            done, ptr = lax.fori_loop(0, NB, block_body,
                                      (jnp.int32(0), jnp.int32(0)))

            @pl.when(done < ptr)
            def _():
                do_batch(done, ptr)
            plsc.subcore_barrier()

            @pl.when(sid < NSUB - 1)
            def _():
                pltpu.sync_copy(acc.at[pl.ds(sid * base, base), :],
                                out_h.at[pl.ds(lo + sid * base, base), :])

            @pl.when(sid == NSUB - 1)
            def _():
                pltpu.sync_copy(
                    acc.at[pl.ds((NSUB - 1) * base, last), :],
                    out_h.at[pl.ds(lo + (NSUB - 1) * base, last), :])
            plsc.subcore_barrier()
            return 0

        lax.fori_loop(0, CPC, chunk_body, 0)

    return spmm


def _spmm(vals, rows, cols, X, n_out, zeros):
    f = _make_spmm(vals.shape[0], n_out, X.shape[0])
    return f(rows.astype(jnp.int32), cols.astype(jnp.int32), vals, X, zeros)


def _dense_body(x_ref, ax_ref, aax_ref, py_ref, w0_ref, w3_ref, w4_ref,
                w5_ref, scale_ref, bias_ref, o_ref):
    h = jnp.dot(x_ref[...], w0_ref[...], preferred_element_type=jnp.float32)
    h += jnp.dot(ax_ref[...], w3_ref[...], preferred_element_type=jnp.float32)
    h += jnp.dot(aax_ref[...], w4_ref[...], preferred_element_type=jnp.float32)
    h += jnp.dot(py_ref[...], w5_ref[...], preferred_element_type=jnp.float32)
    col = lax.broadcasted_iota(jnp.int32, h.shape, 1)
    h = jnp.where(col < D, jnp.maximum(h, 0.0), h)
    o_ref[...] = h * scale_ref[...] + bias_ref[...]


def _dense_stage(x, ax, aax, py, w0, w3, w4, w5, scale, bias, blk):
    n = x.shape[0]
    grid = (n + blk - 1) // blk
    row_spec = pl.BlockSpec((blk, D), lambda i: (i, 0))
    w_spec = pl.BlockSpec((D, 2 * D), lambda i: (0, 0))
    v_spec = pl.BlockSpec((1, 2 * D), lambda i: (0, 0))
    return pl.pallas_call(
        _dense_body,
        grid=(grid,),
        in_specs=[row_spec, row_spec, row_spec, row_spec,
                  w_spec, w_spec, w_spec, w_spec, v_spec, v_spec],
        out_specs=pl.BlockSpec((blk, 2 * D), lambda i: (i, 0)),
        out_shape=jax.ShapeDtypeStruct((n, 2 * D), jnp.float32),
    )(x, ax, aax, py, w0, w3, w4, w5, scale, bias)


def kernel(x, y, adj_val, adjlg_val, p_val, deg, deg_lg, W_alpha, W_beta,
           W_alpha_lg, W_beta_lg, bn_gamma, bn_beta, adj_idx, adjlg_idx,
           p_rows, p_cols):
    N = x.shape[0]
    M = y.shape[0]

    def cat(wa, wb):
        return jnp.concatenate((wa, wb), axis=1)

    w0_n = cat(W_alpha[0] + W_alpha[1] + W_alpha[2],
               W_beta[0] + W_beta[1] + W_beta[2])
    w3_n = cat(W_alpha[3], W_beta[3])
    w4_n = cat(W_alpha[4], W_beta[4])
    w5_n = cat(W_alpha[5], W_beta[5])
    w0_l = cat(W_alpha_lg[0] + W_alpha_lg[1] + W_alpha_lg[2],
               W_beta_lg[0] + W_beta_lg[1] + W_beta_lg[2])
    w3_l = cat(W_alpha_lg[3], W_beta_lg[3])
    w4_l = cat(W_alpha_lg[4], W_beta_lg[4])
    w5_l = w5_n

    inv = 1.0 / jnp.sqrt(1.0 + EPS)
    scale_n = jnp.concatenate((bn_gamma[0], bn_gamma[1]))[None, :] * inv
    bias_n = jnp.concatenate((bn_beta[0], bn_beta[1]))[None, :]
    scale_l = jnp.concatenate((bn_gamma[2], bn_gamma[3]))[None, :] * inv
    bias_l = jnp.concatenate((bn_beta[2], bn_beta[3]))[None, :]

    zeros = jnp.zeros((1024, D), jnp.float32)
    ax = _spmm(adj_val, adj_idx[0], adj_idx[1], x, N, zeros)
    aax = _spmm(adj_val, adj_idx[0], adj_idx[1], ax, N, zeros)
    aly = _spmm(adjlg_val, adjlg_idx[0], adjlg_idx[1], y, M, zeros)
    aaly = _spmm(adjlg_val, adjlg_idx[0], adjlg_idx[1], aly, M, zeros)
    py = _spmm(p_val, p_rows, p_cols, y, N, zeros)
    px = _spmm(p_val, p_cols, p_rows, x, M, zeros)

    out_n = _dense_stage(x, ax, aax, py, w0_n, w3_n, w4_n, w5_n,
                         scale_n, bias_n, blk=512)
    out_l = _dense_stage(y, aly, aaly, px, w0_l, w3_l, w4_l, w5_l,
                         scale_l, bias_l, blk=512)
    return (out_n, out_l)


# speedup vs baseline: 1.4025x; 1.1515x over previous
import functools

import jax
import jax.numpy as jnp
from jax import lax
from jax.experimental import pallas as pl
from jax.experimental.pallas import tpu as pltpu
from jax.experimental.pallas import tpu_sc as plsc

EPS = 1e-3
D = 128
B = 2000
G = 128
CAP = 6144
NSUB = 16
NCORE = 2


_DNUMS = lax.GatherDimensionNumbers(
    offset_dims=(), collapsed_slice_dims=(0,), start_index_map=(0,))


def _vgather(v, idx):
    return lax.gather(v, idx[:, None], _DNUMS, (1,),
                      mode=lax.GatherScatterMode.PROMISE_IN_BOUNDS)


def _lane_bcast(v, j):
    return _vgather(v, jnp.full((16,), j, dtype=jnp.int32))


@functools.cache
def _make_spmm(E, n_out, n_in):
    R = 2000 if n_out <= 10000 else 10000
    C = n_out // R
    assert C * R == n_out
    CPC = (C + NCORE - 1) // NCORE
    E_w = E // NSUB
    NB = E_w // B
    assert NB * B == E_w
    base = (R // NSUB) // 8 * 8
    last = R - (NSUB - 1) * base
    assert last % 8 == 0 and 0 < last

    mesh = plsc.VectorSubcoreMesh(core_axis_name="c", subcore_axis_name="s",
                                  num_cores=NCORE, num_subcores=NSUB)

    @functools.partial(
        pl.kernel,
        out_type=jax.ShapeDtypeStruct((n_out, D), jnp.float32),
        mesh=mesh,
        scratch_types=[
            pltpu.VMEM_SHARED((R, D), jnp.float32),
            pltpu.VMEM((B,), jnp.int32),
            pltpu.VMEM((B,), jnp.int32),
            pltpu.VMEM((B,), jnp.float32),
            pltpu.VMEM((CAP,), jnp.int32),
            pltpu.VMEM((CAP,), jnp.int32),
            pltpu.VMEM((CAP,), jnp.float32),
            pltpu.VMEM((G,), jnp.int32),
            pltpu.VMEM((G,), jnp.int32),
            pltpu.VMEM((G,), jnp.float32),
            pltpu.VMEM((G, D), jnp.float32),
            pltpu.SemaphoreType.DMA,
        ],
    )
    def spmm(rows_h, cols_h, vals_h, x_h, z_h, out_h, acc, rbuf, cbuf, vbuf,
             crows, ccols, cvals, grow, gcol, gval, gbuf, sem):
        cid = lax.axis_index("c")
        sid = lax.axis_index("s")
        lanes = lax.iota(jnp.int32, 16)

        def do_batch(done, ptr):
            def stage_k(k, _):
                offs = done + k * 16
                valid = (offs + lanes) < ptr
                rr = crows[pl.ds(offs, 16)]
                cc = ccols[pl.ds(offs, 16)]
                vv = cvals[pl.ds(offs, 16)]
                grow[pl.ds(k * 16, 16)] = jnp.where(valid, rr, 0)
                gcol[pl.ds(k * 16, 16)] = jnp.where(valid, cc, 0)
                gval[pl.ds(k * 16, 16)] = jnp.where(valid, vv, 0.0)
                return 0
            lax.fori_loop(0, G // 16, stage_k, 0)
            pltpu.async_copy(x_h.at[gcol], gbuf, sem).wait()

            def scale_g(g, _):
                v16 = gval[pl.ds(g * 16, 16)]
                for j in range(16):
                    sp = _lane_bcast(v16, j)
                    e = g * 16 + j
                    for b in range(D // 16):
                        gbuf[e, pl.ds(b * 16, 16)] = (
                            gbuf[e, pl.ds(b * 16, 16)] * sp)
                return 0
            lax.fori_loop(0, G // 16, scale_g, 0)
            pltpu.sync_copy(gbuf, acc.at[grow], add=True)

        def chunk_body(ci, _):
            chunk = ci * NCORE + cid
            lo = chunk * R

            @pl.when(chunk < C)
            def _chunk_work():
                _run_chunk(lo)
            return 0

        def _run_chunk(lo):
            @pl.when(sid < NSUB - 1)
            def _():
                pltpu.sync_copy(z_h.at[pl.ds(0, base), :],
                                acc.at[pl.ds(sid * base, base), :])

            @pl.when(sid == NSUB - 1)
            def _():
                pltpu.sync_copy(z_h.at[pl.ds(0, last), :],
                                acc.at[pl.ds((NSUB - 1) * base, last), :])
            plsc.subcore_barrier()

            def block_body(blk, carry):
                done, ptr = carry
                eb = pl.multiple_of(sid * E_w + blk * B, 16)
                pltpu.sync_copy(rows_h.at[pl.ds(eb, B)], rbuf)
                pltpu.sync_copy(cols_h.at[pl.ds(eb, B)], cbuf)
                pltpu.sync_copy(vals_h.at[pl.ds(eb, B)], vbuf)

                def scan_one(i, ptr):
                    r = rbuf[pl.ds(i * 16, 16)]
                    c = cbuf[pl.ds(i * 16, 16)]
                    v = vbuf[pl.ds(i * 16, 16)]
                    m = (r >= lo) & (r < lo + R)
                    csum = jnp.where(m, 1, 0)
                    for s in (1, 2, 4, 8):
                        sh = _vgather(csum, jnp.maximum(lanes - s, 0))
                        csum = csum + jnp.where(lanes >= s, sh, 0)
                    target = lanes + 1
                    lo_l = jnp.zeros((16,), jnp.int32)
                    hi_l = jnp.full((16,), 15, jnp.int32)
                    for _ in range(4):
                        mid = lax.shift_right_logical(lo_l + hi_l, 1)
                        ge = _vgather(csum, mid) >= target
                        hi_l = jnp.where(ge, mid, hi_l)
                        lo_l = jnp.where(ge, lo_l, mid + 1)
                    crows[pl.ds(ptr, 16)] = _vgather(r - lo, lo_l)
                    ccols[pl.ds(ptr, 16)] = _vgather(c, lo_l)
                    cvals[pl.ds(ptr, 16)] = _vgather(v, lo_l)
                    cnt = jnp.squeeze(lax.slice(csum, (15,), (16,)))
                    ok = jnp.where(ptr < CAP - 32, 1, 0)
                    return ptr + cnt * ok

                def scan_i(i, ptr):
                    ptr = scan_one(i * 2, ptr)
                    return scan_one(i * 2 + 1, ptr)
                ptr = lax.fori_loop(0, B // 32, scan_i, ptr)

                def flush_k(k, c):
                    d, p = c
                    cond = d + G <= p

                    @pl.when(cond)
                    def _():
                        do_batch(d, p)
                    return d + jnp.where(cond, G, 0), p
                done, ptr = lax.fori_loop(0, (B + G - 1) // G + 1, flush_k,
                                          (done, ptr))
                return done, ptr

            done, ptr = lax.fori_loop(0, NB, block_body,
                                      (jnp.int32(0), jnp.int32(0)))

            @pl.when(done < ptr)
            def _():
                do_batch(done, ptr)
            plsc.subcore_barrier()

            @pl.when(sid < NSUB - 1)
            def _():
                pltpu.sync_copy(acc.at[pl.ds(sid * base, base), :],
                                out_h.at[pl.ds(lo + sid * base, base), :])

            @pl.when(sid == NSUB - 1)
            def _():
                pltpu.sync_copy(
                    acc.at[pl.ds((NSUB - 1) * base, last), :],
                    out_h.at[pl.ds(lo + (NSUB - 1) * base, last), :])
            plsc.subcore_barrier()
            return 0

        lax.fori_loop(0, CPC, chunk_body, 0)

    return spmm


def _spmm(vals, rows, cols, X, n_out, zeros):
    f = _make_spmm(vals.shape[0], n_out, X.shape[0])
    return f(rows.astype(jnp.int32), cols.astype(jnp.int32), vals, X, zeros)


def _dense_body(x_ref, ax_ref, aax_ref, py_ref, w0_ref, w3_ref, w4_ref,
                w5_ref, scale_ref, bias_ref, o_ref):
    h = jnp.dot(x_ref[...], w0_ref[...], preferred_element_type=jnp.float32)
    h += jnp.dot(ax_ref[...], w3_ref[...], preferred_element_type=jnp.float32)
    h += jnp.dot(aax_ref[...], w4_ref[...], preferred_element_type=jnp.float32)
    h += jnp.dot(py_ref[...], w5_ref[...], preferred_element_type=jnp.float32)
    col = lax.broadcasted_iota(jnp.int32, h.shape, 1)
    h = jnp.where(col < D, jnp.maximum(h, 0.0), h)
    o_ref[...] = h * scale_ref[...] + bias_ref[...]


def _dense_stage(x, ax, aax, py, w0, w3, w4, w5, scale, bias, blk):
    n = x.shape[0]
    grid = (n + blk - 1) // blk
    row_spec = pl.BlockSpec((blk, D), lambda i: (i, 0))
    w_spec = pl.BlockSpec((D, 2 * D), lambda i: (0, 0))
    v_spec = pl.BlockSpec((1, 2 * D), lambda i: (0, 0))
    return pl.pallas_call(
        _dense_body,
        grid=(grid,),
        in_specs=[row_spec, row_spec, row_spec, row_spec,
                  w_spec, w_spec, w_spec, w_spec, v_spec, v_spec],
        out_specs=pl.BlockSpec((blk, 2 * D), lambda i: (i, 0)),
        out_shape=jax.ShapeDtypeStruct((n, 2 * D), jnp.float32),
    )(x, ax, aax, py, w0, w3, w4, w5, scale, bias)


def kernel(x, y, adj_val, adjlg_val, p_val, deg, deg_lg, W_alpha, W_beta,
           W_alpha_lg, W_beta_lg, bn_gamma, bn_beta, adj_idx, adjlg_idx,
           p_rows, p_cols):
    N = x.shape[0]
    M = y.shape[0]

    def cat(wa, wb):
        return jnp.concatenate((wa, wb), axis=1)

    w0_n = cat(W_alpha[0] + W_alpha[1] + W_alpha[2],
               W_beta[0] + W_beta[1] + W_beta[2])
    w3_n = cat(W_alpha[3], W_beta[3])
    w4_n = cat(W_alpha[4], W_beta[4])
    w5_n = cat(W_alpha[5], W_beta[5])
    w0_l = cat(W_alpha_lg[0] + W_alpha_lg[1] + W_alpha_lg[2],
               W_beta_lg[0] + W_beta_lg[1] + W_beta_lg[2])
    w3_l = cat(W_alpha_lg[3], W_beta_lg[3])
    w4_l = cat(W_alpha_lg[4], W_beta_lg[4])
    w5_l = w5_n

    inv = 1.0 / jnp.sqrt(1.0 + EPS)
    scale_n = jnp.concatenate((bn_gamma[0], bn_gamma[1]))[None, :] * inv
    bias_n = jnp.concatenate((bn_beta[0], bn_beta[1]))[None, :]
    scale_l = jnp.concatenate((bn_gamma[2], bn_gamma[3]))[None, :] * inv
    bias_l = jnp.concatenate((bn_beta[2], bn_beta[3]))[None, :]

    zeros = jnp.zeros((1024, D), jnp.float32)
    ax = _spmm(adj_val, adj_idx[0], adj_idx[1], x, N, zeros)
    aax = _spmm(adj_val, adj_idx[0], adj_idx[1], ax, N, zeros)
    aly = _spmm(adjlg_val, adjlg_idx[0], adjlg_idx[1], y, M, zeros)
    aaly = _spmm(adjlg_val, adjlg_idx[0], adjlg_idx[1], aly, M, zeros)
    py = _spmm(p_val, p_rows, p_cols, y, N, zeros)
    px = _spmm(p_val, p_cols, p_rows, x, M, zeros)

    out_n = _dense_stage(x, ax, aax, py, w0_n, w3_n, w4_n, w5_n,
                         scale_n, bias_n, blk=512)
    out_l = _dense_stage(y, aly, aaly, px, w0_l, w3_l, w4_l, w5_l,
                         scale_l, bias_l, blk=512)
    return (out_n, out_l)


# speedup vs baseline: 1.4230x; 1.0146x over previous
import functools

import jax
import jax.numpy as jnp
from jax import lax
from jax.experimental import pallas as pl
from jax.experimental.pallas import tpu as pltpu
from jax.experimental.pallas import tpu_sc as plsc

EPS = 1e-3
D = 128
B = 2000
G = 128
CAP = 6144
NSUB = 16
NCORE = 2


_DNUMS = lax.GatherDimensionNumbers(
    offset_dims=(), collapsed_slice_dims=(0,), start_index_map=(0,))


def _vgather(v, idx):
    return lax.gather(v, idx[:, None], _DNUMS, (1,),
                      mode=lax.GatherScatterMode.PROMISE_IN_BOUNDS)


def _lane_bcast(v, j):
    return _vgather(v, jnp.full((16,), j, dtype=jnp.int32))


@functools.cache
def _make_spmm(E, n_out, n_in):
    R = 2000 if n_out <= 10000 else 10000
    C = n_out // R
    assert C * R == n_out
    CPC = (C + NCORE - 1) // NCORE
    E_w = E // NSUB
    NB = E_w // B
    assert NB * B == E_w
    base = (R // NSUB) // 8 * 8
    last = R - (NSUB - 1) * base
    assert last % 8 == 0 and 0 < last

    mesh = plsc.VectorSubcoreMesh(core_axis_name="c", subcore_axis_name="s",
                                  num_cores=NCORE, num_subcores=NSUB)

    @functools.partial(
        pl.kernel,
        out_type=jax.ShapeDtypeStruct((n_out, D), jnp.float32),
        mesh=mesh,
        scratch_types=[
            pltpu.VMEM_SHARED((R, D), jnp.float32),
            pltpu.VMEM((B,), jnp.int32),
            pltpu.VMEM((B,), jnp.int32),
            pltpu.VMEM((B,), jnp.float32),
            pltpu.VMEM((CAP,), jnp.int32),
            pltpu.VMEM((CAP,), jnp.int32),
            pltpu.VMEM((CAP,), jnp.float32),
            pltpu.VMEM((G,), jnp.int32),
            pltpu.VMEM((G,), jnp.int32),
            pltpu.VMEM((G,), jnp.float32),
            pltpu.VMEM((G, D), jnp.float32),
            pltpu.SemaphoreType.DMA,
        ],
    )
    def spmm(rows_h, cols_h, vals_h, x_h, z_h, out_h, acc, rbuf, cbuf, vbuf,
             crows, ccols, cvals, grow, gcol, gval, gbuf, sem):
        cid = lax.axis_index("c")
        sid = lax.axis_index("s")
        lanes = lax.iota(jnp.int32, 16)

        def do_batch(done, ptr):
            def stage_k(k, _):
                offs = done + k * 16
                valid = (offs + lanes) < ptr
                rr = crows[pl.ds(offs, 16)]
                cc = ccols[pl.ds(offs, 16)]
                vv = cvals[pl.ds(offs, 16)]
                grow[pl.ds(k * 16, 16)] = jnp.where(valid, rr, 0)
                gcol[pl.ds(k * 16, 16)] = jnp.where(valid, cc, 0)
                gval[pl.ds(k * 16, 16)] = jnp.where(valid, vv, 0.0)
                return 0
            lax.fori_loop(0, G // 16, stage_k, 0)
            pltpu.async_copy(x_h.at[gcol], gbuf, sem).wait()

            def scale_g(g, _):
                v16 = gval[pl.ds(g * 16, 16)]
                for j in range(16):
                    sp = _lane_bcast(v16, j)
                    e = g * 16 + j
                    for b in range(D // 16):
                        gbuf[e, pl.ds(b * 16, 16)] = (
                            gbuf[e, pl.ds(b * 16, 16)] * sp)
                return 0
            lax.fori_loop(0, G // 16, scale_g, 0)
            pltpu.sync_copy(gbuf, acc.at[grow], add=True)

        def chunk_body(ci, _):
            chunk = ci * NCORE + cid
            lo = chunk * R

            @pl.when(chunk < C)
            def _chunk_work():
                _run_chunk(lo)
            return 0

        def _run_chunk(lo):
            @pl.when(sid < NSUB - 1)
            def _():
                pltpu.sync_copy(z_h.at[pl.ds(0, base), :],
                                acc.at[pl.ds(sid * base, base), :])

            @pl.when(sid == NSUB - 1)
            def _():
                pltpu.sync_copy(z_h.at[pl.ds(0, last), :],
                                acc.at[pl.ds((NSUB - 1) * base, last), :])
            plsc.subcore_barrier()

            def block_body(blk, carry):
                done, ptr = carry
                eb = pl.multiple_of(sid * E_w + blk * B, 16)
                pltpu.sync_copy(rows_h.at[pl.ds(eb, B)], rbuf)
                pltpu.sync_copy(cols_h.at[pl.ds(eb, B)], cbuf)
                pltpu.sync_copy(vals_h.at[pl.ds(eb, B)], vbuf)

                def scan_one(i, ptr):
                    r = rbuf[pl.ds(i * 16, 16)]
                    c = cbuf[pl.ds(i * 16, 16)]
                    v = vbuf[pl.ds(i * 16, 16)]
                    m = (r >= lo) & (r < lo + R)
                    csum = jnp.where(m, 1, 0)
                    for s in (1, 2, 4, 8):
                        sh = _vgather(csum, jnp.maximum(lanes - s, 0))
                        csum = csum + jnp.where(lanes >= s, sh, 0)
                    target = lanes + 1
                    lo_l = jnp.zeros((16,), jnp.int32)
                    hi_l = jnp.full((16,), 15, jnp.int32)
                    for _ in range(4):
                        mid = lax.shift_right_logical(lo_l + hi_l, 1)
                        ge = _vgather(csum, mid) >= target
                        hi_l = jnp.where(ge, mid, hi_l)
                        lo_l = jnp.where(ge, lo_l, mid + 1)
                    crows[pl.ds(ptr, 16)] = _vgather(r - lo, lo_l)
                    ccols[pl.ds(ptr, 16)] = _vgather(c, lo_l)
                    cvals[pl.ds(ptr, 16)] = _vgather(v, lo_l)
                    cnt = jnp.squeeze(lax.slice(csum, (15,), (16,)))
                    ok = jnp.where(ptr < CAP - 32, 1, 0)
                    return ptr + cnt * ok

                def scan_i(i, ptr):
                    ptr = scan_one(i * 2, ptr)
                    return scan_one(i * 2 + 1, ptr)
                ptr = lax.fori_loop(0, B // 32, scan_i, ptr)
                for t in range(B // 32 * 2, B // 16):
                    ptr = scan_one(t, ptr)

                def flush_k(k, c):
                    d, p = c
                    cond = d + G <= p

                    @pl.when(cond)
                    def _():
                        do_batch(d, p)
                    return d + jnp.where(cond, G, 0), p
                done, ptr = lax.fori_loop(0, (B + G - 1) // G + 1, flush_k,
                                          (done, ptr))
                return done, ptr

            done, ptr = lax.fori_loop(0, NB, block_body,
                                      (jnp.int32(0), jnp.int32(0)))

            @pl.when(done < ptr)
            def _():
                do_batch(done, ptr)
            plsc.subcore_barrier()

            @pl.when(sid < NSUB - 1)
            def _():
                pltpu.sync_copy(acc.at[pl.ds(sid * base, base), :],
                                out_h.at[pl.ds(lo + sid * base, base), :])

            @pl.when(sid == NSUB - 1)
            def _():
                pltpu.sync_copy(
                    acc.at[pl.ds((NSUB - 1) * base, last), :],
                    out_h.at[pl.ds(lo + (NSUB - 1) * base, last), :])
            plsc.subcore_barrier()
            return 0

        lax.fori_loop(0, CPC, chunk_body, 0)

    return spmm


def _spmm(vals, rows, cols, X, n_out, zeros):
    f = _make_spmm(vals.shape[0], n_out, X.shape[0])
    return f(rows.astype(jnp.int32), cols.astype(jnp.int32), vals, X, zeros)


def _dense_body(x_ref, ax_ref, aax_ref, py_ref, w0_ref, w3_ref, w4_ref,
                w5_ref, scale_ref, bias_ref, o_ref):
    h = jnp.dot(x_ref[...], w0_ref[...], preferred_element_type=jnp.float32)
    h += jnp.dot(ax_ref[...], w3_ref[...], preferred_element_type=jnp.float32)
    h += jnp.dot(aax_ref[...], w4_ref[...], preferred_element_type=jnp.float32)
    h += jnp.dot(py_ref[...], w5_ref[...], preferred_element_type=jnp.float32)
    col = lax.broadcasted_iota(jnp.int32, h.shape, 1)
    h = jnp.where(col < D, jnp.maximum(h, 0.0), h)
    o_ref[...] = h * scale_ref[...] + bias_ref[...]


def _dense_stage(x, ax, aax, py, w0, w3, w4, w5, scale, bias, blk):
    n = x.shape[0]
    grid = (n + blk - 1) // blk
    row_spec = pl.BlockSpec((blk, D), lambda i: (i, 0))
    w_spec = pl.BlockSpec((D, 2 * D), lambda i: (0, 0))
    v_spec = pl.BlockSpec((1, 2 * D), lambda i: (0, 0))
    return pl.pallas_call(
        _dense_body,
        grid=(grid,),
        in_specs=[row_spec, row_spec, row_spec, row_spec,
                  w_spec, w_spec, w_spec, w_spec, v_spec, v_spec],
        out_specs=pl.BlockSpec((blk, 2 * D), lambda i: (i, 0)),
        out_shape=jax.ShapeDtypeStruct((n, 2 * D), jnp.float32),
    )(x, ax, aax, py, w0, w3, w4, w5, scale, bias)


def kernel(x, y, adj_val, adjlg_val, p_val, deg, deg_lg, W_alpha, W_beta,
           W_alpha_lg, W_beta_lg, bn_gamma, bn_beta, adj_idx, adjlg_idx,
           p_rows, p_cols):
    N = x.shape[0]
    M = y.shape[0]

    def cat(wa, wb):
        return jnp.concatenate((wa, wb), axis=1)

    w0_n = cat(W_alpha[0] + W_alpha[1] + W_alpha[2],
               W_beta[0] + W_beta[1] + W_beta[2])
    w3_n = cat(W_alpha[3], W_beta[3])
    w4_n = cat(W_alpha[4], W_beta[4])
    w5_n = cat(W_alpha[5], W_beta[5])
    w0_l = cat(W_alpha_lg[0] + W_alpha_lg[1] + W_alpha_lg[2],
               W_beta_lg[0] + W_beta_lg[1] + W_beta_lg[2])
    w3_l = cat(W_alpha_lg[3], W_beta_lg[3])
    w4_l = cat(W_alpha_lg[4], W_beta_lg[4])
    w5_l = w5_n

    inv = 1.0 / jnp.sqrt(1.0 + EPS)
    scale_n = jnp.concatenate((bn_gamma[0], bn_gamma[1]))[None, :] * inv
    bias_n = jnp.concatenate((bn_beta[0], bn_beta[1]))[None, :]
    scale_l = jnp.concatenate((bn_gamma[2], bn_gamma[3]))[None, :] * inv
    bias_l = jnp.concatenate((bn_beta[2], bn_beta[3]))[None, :]

    zeros = jnp.zeros((1024, D), jnp.float32)
    ax = _spmm(adj_val, adj_idx[0], adj_idx[1], x, N, zeros)
    aax = _spmm(adj_val, adj_idx[0], adj_idx[1], ax, N, zeros)
    aly = _spmm(adjlg_val, adjlg_idx[0], adjlg_idx[1], y, M, zeros)
    aaly = _spmm(adjlg_val, adjlg_idx[0], adjlg_idx[1], aly, M, zeros)
    py = _spmm(p_val, p_rows, p_cols, y, N, zeros)
    px = _spmm(p_val, p_cols, p_rows, x, M, zeros)

    out_n = _dense_stage(x, ax, aax, py, w0_n, w3_n, w4_n, w5_n,
                         scale_n, bias_n, blk=512)
    out_l = _dense_stage(y, aly, aaly, px, w0_l, w3_l, w4_l, w5_l,
                         scale_l, bias_l, blk=512)
    return (out_n, out_l)
